# baseline (speedup 1.0000x reference)
import jax
import jax.numpy as jnp
from jax.experimental import pallas as pl


def _fused_gcn_kernel(x_ref, a_ref, wg_ref, bg_ref, wd_ref, bd_ref, out_ref):
    x_b = x_ref[0]
    h = jax.lax.dot_general(
        x_b, wg_ref[...],
        dimension_numbers=(((0,), (0,)), ((), ())),
        preferred_element_type=jnp.float32,
    )
    g = jnp.dot(a_ref[...], h, preferred_element_type=jnp.float32)
    g = jnp.maximum(g + bg_ref[...], 0.0)
    o = jnp.dot(g, wd_ref[...], preferred_element_type=jnp.float32)
    out_ref[0] = jnp.maximum(o + bd_ref[...], 0.0)


def kernel(x, a, W_gcn, b_gcn, W_d, b_d):
    B, T, N = x.shape
    H = W_gcn.shape[1]
    bg = b_gcn.reshape(1, H)
    bd = b_d.reshape(1, T)
    return pl.pallas_call(
        _fused_gcn_kernel,
        grid=(B,),
        in_specs=[
            pl.BlockSpec((1, T, N), lambda b: (b, 0, 0)),
            pl.BlockSpec((N, N), lambda b: (0, 0)),
            pl.BlockSpec((T, H), lambda b: (0, 0)),
            pl.BlockSpec((1, H), lambda b: (0, 0)),
            pl.BlockSpec((H, T), lambda b: (0, 0)),
            pl.BlockSpec((1, T), lambda b: (0, 0)),
        ],
        out_specs=pl.BlockSpec((1, N, T), lambda b: (b, 0, 0)),
        out_shape=jax.ShapeDtypeStruct((B, N, T), jnp.float32),
    )(x, a, W_gcn, bg, W_d, bd)


# speedup vs baseline: 1.4400x; 1.4400x over previous
import jax
import jax.numpy as jnp
from jax.experimental import pallas as pl


def _stage1_kernel(x_ref, wg_ref, hall_ref):
    h0 = jax.lax.dot_general(
        x_ref[0], wg_ref[...],
        dimension_numbers=(((0,), (0,)), ((), ())),
        preferred_element_type=jnp.float32,
    )
    h1 = jax.lax.dot_general(
        x_ref[1], wg_ref[...],
        dimension_numbers=(((0,), (0,)), ((), ())),
        preferred_element_type=jnp.float32,
    )
    hall_ref[...] = jnp.concatenate([h0, h1], axis=1)


def _stage2_kernel(a_ref, hall_ref, bg_ref, wd_ref, bd_ref, out_ref, *, B, H):
    g = jnp.dot(a_ref[...], hall_ref[...], preferred_element_type=jnp.float32)
    for b in range(B):
        gb = jnp.maximum(g[:, b * H:(b + 1) * H] + bg_ref[...], 0.0)
        ob = jnp.dot(gb, wd_ref[...], preferred_element_type=jnp.float32)
        out_ref[b] = jnp.maximum(ob + bd_ref[...], 0.0)


def kernel(x, a, W_gcn, b_gcn, W_d, b_d):
    B, T, N = x.shape
    H = W_gcn.shape[1]
    bg = b_gcn.reshape(1, H)
    bd = b_d.reshape(1, T)

    hall = pl.pallas_call(
        _stage1_kernel,
        grid=(B // 2,),
        in_specs=[
            pl.BlockSpec((2, T, N), lambda b: (b, 0, 0)),
            pl.BlockSpec((T, H), lambda b: (0, 0)),
        ],
        out_specs=pl.BlockSpec((N, 2 * H), lambda b: (0, b)),
        out_shape=jax.ShapeDtypeStruct((N, B * H), jnp.float32),
    )(x, W_gcn)

    BA = 256
    import functools
    return pl.pallas_call(
        functools.partial(_stage2_kernel, B=B, H=H),
        grid=(N // BA,),
        in_specs=[
            pl.BlockSpec((BA, N), lambda i: (i, 0)),
            pl.BlockSpec((N, B * H), lambda i: (0, 0)),
            pl.BlockSpec((1, H), lambda i: (0, 0)),
            pl.BlockSpec((H, T), lambda i: (0, 0)),
            pl.BlockSpec((1, T), lambda i: (0, 0)),
        ],
        out_specs=pl.BlockSpec((B, BA, T), lambda i: (0, i, 0)),
        out_shape=jax.ShapeDtypeStruct((B, N, T), jnp.float32),
    )(a, hall, bg, W_d, bd)


# speedup vs baseline: 1.4649x; 1.0173x over previous
import jax
import jax.numpy as jnp
from jax.experimental import pallas as pl


def _stage1_kernel(x_ref, wg_ref, hall_ref):
    wg = wg_ref[...].astype(jnp.bfloat16)
    h0 = jax.lax.dot_general(
        x_ref[0].astype(jnp.bfloat16), wg,
        dimension_numbers=(((0,), (0,)), ((), ())),
        preferred_element_type=jnp.float32,
    )
    h1 = jax.lax.dot_general(
        x_ref[1].astype(jnp.bfloat16), wg,
        dimension_numbers=(((0,), (0,)), ((), ())),
        preferred_element_type=jnp.float32,
    )
    hall_ref[...] = jnp.concatenate([h0, h1], axis=1)


def _stage2_kernel(a_ref, hall_ref, bg_ref, wd_ref, bd_ref, out_ref, *, B, H):
    g = jnp.dot(a_ref[...].astype(jnp.bfloat16),
                hall_ref[...].astype(jnp.bfloat16),
                preferred_element_type=jnp.float32)
    wd = wd_ref[...].astype(jnp.bfloat16)
    for b in range(B):
        gb = jnp.maximum(g[:, b * H:(b + 1) * H] + bg_ref[...], 0.0)
        ob = jnp.dot(gb.astype(jnp.bfloat16), wd,
                     preferred_element_type=jnp.float32)
        out_ref[b] = jnp.maximum(ob + bd_ref[...], 0.0)


def kernel(x, a, W_gcn, b_gcn, W_d, b_d):
    B, T, N = x.shape
    H = W_gcn.shape[1]
    bg = b_gcn.reshape(1, H)
    bd = b_d.reshape(1, T)

    hall = pl.pallas_call(
        _stage1_kernel,
        grid=(B // 2,),
        in_specs=[
            pl.BlockSpec((2, T, N), lambda b: (b, 0, 0)),
            pl.BlockSpec((T, H), lambda b: (0, 0)),
        ],
        out_specs=pl.BlockSpec((N, 2 * H), lambda b: (0, b)),
        out_shape=jax.ShapeDtypeStruct((N, B * H), jnp.float32),
    )(x, W_gcn)

    BA = 256
    import functools
    return pl.pallas_call(
        functools.partial(_stage2_kernel, B=B, H=H),
        grid=(N // BA,),
        in_specs=[
            pl.BlockSpec((BA, N), lambda i: (i, 0)),
            pl.BlockSpec((N, B * H), lambda i: (0, 0)),
            pl.BlockSpec((1, H), lambda i: (0, 0)),
            pl.BlockSpec((H, T), lambda i: (0, 0)),
            pl.BlockSpec((1, T), lambda i: (0, 0)),
        ],
        out_specs=pl.BlockSpec((B, BA, T), lambda i: (0, i, 0)),
        out_shape=jax.ShapeDtypeStruct((B, N, T), jnp.float32),
    )(a, hall, bg, W_d, bd)
